# baseline (speedup 1.0000x reference)
import functools

import jax
import jax.numpy as jnp
from jax import lax
from jax.experimental import pallas as pl
from jax.experimental.pallas import tpu as pltpu
from jax.experimental.pallas import tpu_sc as plsc

N = 10000
E = 320000
D = 128
G = 64
NC = 2
NS = 16
NW = NC * NS
CHUNK = 125
CPW = E // (NW * CHUNK)
NGROUPS = 2
GS = CPW // NGROUPS
ROWS2D = E // CHUNK
NPAD = 10112
RPS = NPAD // NS

BLK = 2000
NB = N // BLK


def _sc_agg_body(h_hbm, src_hbm, dst_hbm, zeros_hbm, out_hbm,
                 src_v, dst_v, rows0_v, rows1_v, agg_sh,
                 gsem0, gsem1, ssem0, ssem1):
    c = lax.axis_index("c")
    s = lax.axis_index("s")
    wid = c * NS + s
    rows = (rows0_v, rows1_v)
    gsems = (gsem0, gsem1)
    ssems = (ssem0, ssem1)

    pltpu.sync_copy(zeros_hbm.at[pl.ds(s * RPS, RPS)],
                    agg_sh.at[pl.ds(s * RPS, RPS)])
    plsc.subcore_barrier()

    for g in range(NGROUPS):
        pltpu.sync_copy(src_hbm.at[pl.ds(wid * CPW + g * GS, GS)], src_v)
        pltpu.sync_copy(dst_hbm.at[pl.ds(wid * CPW + g * GS, GS)], dst_v)
        pltpu.async_copy(h_hbm.at[src_v.at[0]], rows0_v, gsem0)
        pltpu.async_copy(h_hbm.at[src_v.at[1]], rows1_v, gsem1)

        @pl.loop(0, GS, step=2)
        def _(j):
            for b in range(2):
                k = j + b
                pltpu.make_async_copy(h_hbm.at[src_v.at[k]], rows[b],
                                      gsems[b]).wait()
                pltpu.async_copy(rows[b], agg_sh.at[dst_v.at[k]], ssems[b],
                                 add=True)

                @pl.when(jnp.logical_and(k >= 1, k + 1 < GS))
                def _():
                    pltpu.make_async_copy(
                        rows[1 - b], agg_sh.at[dst_v.at[k - 1]],
                        ssems[1 - b]).wait()
                    pltpu.async_copy(h_hbm.at[src_v.at[k + 1]], rows[1 - b],
                                     gsems[1 - b])

        pltpu.make_async_copy(rows[0], agg_sh.at[dst_v.at[GS - 2]],
                              ssems[0]).wait()
        pltpu.make_async_copy(rows[1], agg_sh.at[dst_v.at[GS - 1]],
                              ssems[1]).wait()

    plsc.subcore_barrier()

    pltpu.sync_copy(agg_sh.at[pl.ds(s * RPS, RPS)],
                    out_hbm.at[c, pl.ds(s * RPS, RPS)])


@functools.cache
def _get_sc_agg():
    return pl.kernel(
        _sc_agg_body,
        out_type=jax.ShapeDtypeStruct((NC, NPAD, D), jnp.float32),
        mesh=plsc.VectorSubcoreMesh(core_axis_name="c", subcore_axis_name="s",
                                    num_cores=NC, num_subcores=NS),
        scratch_types=[
            pltpu.VMEM((GS, CHUNK), jnp.int32),
            pltpu.VMEM((GS, CHUNK), jnp.int32),
            pltpu.VMEM((CHUNK, D), jnp.float32),
            pltpu.VMEM((CHUNK, D), jnp.float32),
            pltpu.VMEM_SHARED((NPAD, D), jnp.float32),
            pltpu.SemaphoreType.DMA,
            pltpu.SemaphoreType.DMA,
            pltpu.SemaphoreType.DMA,
            pltpu.SemaphoreType.DMA,
        ],
    )


def _layer_body(h_ref, a_ref, w_ref, b_ref, o_ref):
    acc = h_ref[...] + a_ref[0] + a_ref[1]
    o_ref[...] = jnp.maximum(
        jnp.dot(acc, w_ref[...], preferred_element_type=jnp.float32)
        + b_ref[...], 0.0)


_tc_layer = pl.pallas_call(
    _layer_body,
    grid=(NB,),
    in_specs=[
        pl.BlockSpec((BLK, D), lambda i: (i, 0)),
        pl.BlockSpec((NC, BLK, D), lambda i: (0, i, 0)),
        pl.BlockSpec((D, D), lambda i: (0, 0)),
        pl.BlockSpec((1, D), lambda i: (0, 0)),
    ],
    out_specs=pl.BlockSpec((BLK, D), lambda i: (i, 0)),
    out_shape=jax.ShapeDtypeStruct((N, D), jnp.float32),
)


def _layer3_pool_body(h_ref, a_ref, w_ref, b_ref, batch_ref, wm_ref, bm_ref,
                      o_ref, acc_ref, cnt_ref):
    i = pl.program_id(0)

    @pl.when(i == 0)
    def _():
        acc_ref[...] = jnp.zeros_like(acc_ref)
        cnt_ref[...] = jnp.zeros_like(cnt_ref)

    acc = h_ref[...] + a_ref[0] + a_ref[1]
    h3 = jnp.maximum(
        jnp.dot(acc, w_ref[...], preferred_element_type=jnp.float32)
        + b_ref[...], 0.0)

    bvec = batch_ref[0, 0, :]
    seg = lax.broadcasted_iota(jnp.int32, (G, BLK), 0)
    onehot_t = (bvec[None, :] == seg).astype(jnp.float32)
    acc_ref[...] += jnp.dot(onehot_t, h3,
                            preferred_element_type=jnp.float32)
    cnt_ref[...] += jnp.sum(onehot_t, axis=1, keepdims=True)

    @pl.when(i == NB - 1)
    def _():
        pooled = acc_ref[...] / jnp.maximum(cnt_ref[...], 1.0)
        o_ref[...] = (jnp.dot(pooled, wm_ref[...],
                              preferred_element_type=jnp.float32)
                      + bm_ref[...])


_tc_layer3_pool = pl.pallas_call(
    _layer3_pool_body,
    grid=(NB,),
    in_specs=[
        pl.BlockSpec((BLK, D), lambda i: (i, 0)),
        pl.BlockSpec((NC, BLK, D), lambda i: (0, i, 0)),
        pl.BlockSpec((D, D), lambda i: (0, 0)),
        pl.BlockSpec((1, D), lambda i: (0, 0)),
        pl.BlockSpec((1, 1, BLK), lambda i: (i, 0, 0)),
        pl.BlockSpec((D, D), lambda i: (0, 0)),
        pl.BlockSpec((1, D), lambda i: (0, 0)),
    ],
    out_specs=pl.BlockSpec((G, D), lambda i: (0, 0)),
    out_shape=jax.ShapeDtypeStruct((G, D), jnp.float32),
    scratch_shapes=[
        pltpu.VMEM((G, D), jnp.float32),
        pltpu.VMEM((G, 1), jnp.float32),
    ],
)


def kernel(x, edge_index, batch, W1, b1, W2, b2, W3, b3, Wm, bm):
    src2d = edge_index[0].reshape(ROWS2D, CHUNK)
    dst2d = edge_index[1].reshape(ROWS2D, CHUNK)
    zeros = jnp.zeros((NPAD, D), jnp.float32)

    h = x
    for (W, b) in ((W1, b1), (W2, b2)):
        agg2 = _get_sc_agg()(h, src2d, dst2d, zeros)
        h = _tc_layer(h, agg2, W, b.reshape(1, D))
    agg2 = _get_sc_agg()(h, src2d, dst2d, zeros)
    return _tc_layer3_pool(h, agg2, W3, b3.reshape(1, D),
                           batch.reshape(NB, 1, BLK), Wm, bm.reshape(1, D))


# speedup vs baseline: 1.0295x; 1.0295x over previous
import functools

import jax
import jax.numpy as jnp
from jax import lax
from jax.experimental import pallas as pl
from jax.experimental.pallas import tpu as pltpu
from jax.experimental.pallas import tpu_sc as plsc

N = 10000
E = 320000
D = 128
G = 64
NC = 2
NS = 16
NW = NC * NS
CHUNK = 125
CPW = E // (NW * CHUNK)
NGROUPS = 2
GS = CPW // NGROUPS
ROWS2D = E // CHUNK
NPAD = 10112
RPS = NPAD // NS

BLK = 2000
NB = N // BLK


def _sc_agg_body(h_hbm, src_hbm, dst_hbm, zeros_hbm, out_hbm,
                 src_v, dst_v, rows0_v, rows1_v, agg_sh,
                 gsem0, gsem1, ssem0, ssem1):
    c = lax.axis_index("c")
    s = lax.axis_index("s")
    wid = c * NS + s
    rows = (rows0_v, rows1_v)
    gsems = (gsem0, gsem1)
    ssems = (ssem0, ssem1)

    zc = pltpu.async_copy(zeros_hbm.at[pl.ds(s * RPS, RPS)],
                          agg_sh.at[pl.ds(s * RPS, RPS)], ssem0)
    sc0 = pltpu.async_copy(src_hbm.at[pl.ds(wid * CPW, GS)], src_v, ssem1)
    dc0 = pltpu.async_copy(dst_hbm.at[pl.ds(wid * CPW, GS)], dst_v, gsem1)
    sc0.wait()
    dc0.wait()
    pltpu.async_copy(h_hbm.at[src_v.at[0]], rows0_v, gsem0)
    zc.wait()
    plsc.subcore_barrier()

    for g in range(NGROUPS):
        if g > 0:
            pltpu.sync_copy(src_hbm.at[pl.ds(wid * CPW + g * GS, GS)], src_v)
            pltpu.sync_copy(dst_hbm.at[pl.ds(wid * CPW + g * GS, GS)], dst_v)
            pltpu.async_copy(h_hbm.at[src_v.at[0]], rows0_v, gsem0)

        @pl.loop(0, GS, step=2)
        def _(j):
            for b in range(2):
                je = j + b
                pltpu.make_async_copy(h_hbm.at[src_v.at[je]], rows[b],
                                      gsems[b]).wait()
                nxt = je + 1

                @pl.when(nxt < GS)
                def _():
                    pltpu.async_copy(h_hbm.at[src_v.at[nxt]], rows[1 - b],
                                     gsems[1 - b])

                pltpu.sync_copy(rows[b], agg_sh.at[dst_v.at[je]], add=True)

    plsc.subcore_barrier()

    pltpu.sync_copy(agg_sh.at[pl.ds(s * RPS, RPS)],
                    out_hbm.at[c, pl.ds(s * RPS, RPS)])


@functools.cache
def _get_sc_agg():
    return pl.kernel(
        _sc_agg_body,
        out_type=jax.ShapeDtypeStruct((NC, NPAD, D), jnp.float32),
        mesh=plsc.VectorSubcoreMesh(core_axis_name="c", subcore_axis_name="s",
                                    num_cores=NC, num_subcores=NS),
        scratch_types=[
            pltpu.VMEM((GS, CHUNK), jnp.int32),
            pltpu.VMEM((GS, CHUNK), jnp.int32),
            pltpu.VMEM((CHUNK, D), jnp.float32),
            pltpu.VMEM((CHUNK, D), jnp.float32),
            pltpu.VMEM_SHARED((NPAD, D), jnp.float32),
            pltpu.SemaphoreType.DMA,
            pltpu.SemaphoreType.DMA,
            pltpu.SemaphoreType.DMA,
            pltpu.SemaphoreType.DMA,
        ],
    )


def _layer_body(h_ref, a_ref, w_ref, b_ref, o_ref):
    acc = h_ref[...] + a_ref[0] + a_ref[1]
    o_ref[...] = jnp.maximum(
        jnp.dot(acc, w_ref[...], preferred_element_type=jnp.float32)
        + b_ref[...], 0.0)


_tc_layer = pl.pallas_call(
    _layer_body,
    grid=(NB,),
    in_specs=[
        pl.BlockSpec((BLK, D), lambda i: (i, 0)),
        pl.BlockSpec((NC, BLK, D), lambda i: (0, i, 0)),
        pl.BlockSpec((D, D), lambda i: (0, 0)),
        pl.BlockSpec((1, D), lambda i: (0, 0)),
    ],
    out_specs=pl.BlockSpec((BLK, D), lambda i: (i, 0)),
    out_shape=jax.ShapeDtypeStruct((N, D), jnp.float32),
)


def _layer3_pool_body(h_ref, a_ref, w_ref, b_ref, batch_ref, wm_ref, bm_ref,
                      o_ref, acc_ref, cnt_ref):
    i = pl.program_id(0)

    @pl.when(i == 0)
    def _():
        acc_ref[...] = jnp.zeros_like(acc_ref)
        cnt_ref[...] = jnp.zeros_like(cnt_ref)

    acc = h_ref[...] + a_ref[0] + a_ref[1]
    h3 = jnp.maximum(
        jnp.dot(acc, w_ref[...], preferred_element_type=jnp.float32)
        + b_ref[...], 0.0)

    bvec = batch_ref[0, 0, :]
    seg = lax.broadcasted_iota(jnp.int32, (G, BLK), 0)
    onehot_t = (bvec[None, :] == seg).astype(jnp.float32)
    acc_ref[...] += jnp.dot(onehot_t, h3,
                            preferred_element_type=jnp.float32)
    cnt_ref[...] += jnp.sum(onehot_t, axis=1, keepdims=True)

    @pl.when(i == NB - 1)
    def _():
        pooled = acc_ref[...] / jnp.maximum(cnt_ref[...], 1.0)
        o_ref[...] = (jnp.dot(pooled, wm_ref[...],
                              preferred_element_type=jnp.float32)
                      + bm_ref[...])


_tc_layer3_pool = pl.pallas_call(
    _layer3_pool_body,
    grid=(NB,),
    in_specs=[
        pl.BlockSpec((BLK, D), lambda i: (i, 0)),
        pl.BlockSpec((NC, BLK, D), lambda i: (0, i, 0)),
        pl.BlockSpec((D, D), lambda i: (0, 0)),
        pl.BlockSpec((1, D), lambda i: (0, 0)),
        pl.BlockSpec((1, 1, BLK), lambda i: (i, 0, 0)),
        pl.BlockSpec((D, D), lambda i: (0, 0)),
        pl.BlockSpec((1, D), lambda i: (0, 0)),
    ],
    out_specs=pl.BlockSpec((G, D), lambda i: (0, 0)),
    out_shape=jax.ShapeDtypeStruct((G, D), jnp.float32),
    scratch_shapes=[
        pltpu.VMEM((G, D), jnp.float32),
        pltpu.VMEM((G, 1), jnp.float32),
    ],
)


def kernel(x, edge_index, batch, W1, b1, W2, b2, W3, b3, Wm, bm):
    src2d = edge_index[0].reshape(ROWS2D, CHUNK)
    dst2d = edge_index[1].reshape(ROWS2D, CHUNK)
    zeros = jnp.zeros((NPAD, D), jnp.float32)

    h = x
    for (W, b) in ((W1, b1), (W2, b2)):
        agg2 = _get_sc_agg()(h, src2d, dst2d, zeros)
        h = _tc_layer(h, agg2, W, b.reshape(1, D))
    agg2 = _get_sc_agg()(h, src2d, dst2d, zeros)
    return _tc_layer3_pool(h, agg2, W3, b3.reshape(1, D),
                           batch.reshape(NB, 1, BLK), Wm, bm.reshape(1, D))


# speedup vs baseline: 1.0602x; 1.0298x over previous
import functools

import jax
import jax.numpy as jnp
from jax import lax
from jax.experimental import pallas as pl
from jax.experimental.pallas import tpu as pltpu
from jax.experimental.pallas import tpu_sc as plsc

N = 10000
E = 320000
D = 128
G = 64
NC = 2
NS = 16
NW = NC * NS
CHUNK = 125
CPW = E // (NW * CHUNK)
NGROUPS = 2
GS = CPW // NGROUPS
ROWS2D = E // CHUNK
NPAD = 10112
RPS = NPAD // NS

BLK = 2000
NB = N // BLK


def _sc_agg_body(h_hbm, edges_hbm, zeros_hbm, out_hbm,
                 src_v, dst_v, rows0_v, rows1_v, agg_sh,
                 gsem0, gsem1, ssem0, ssem1):
    c = lax.axis_index("c")
    s = lax.axis_index("s")
    wid = c * NS + s
    rows = (rows0_v, rows1_v)
    gsems = (gsem0, gsem1)
    ssems = (ssem0, ssem1)

    zc = pltpu.async_copy(zeros_hbm.at[pl.ds(s * RPS, RPS)],
                          agg_sh.at[pl.ds(s * RPS, RPS)], ssem0)
    sc0 = pltpu.async_copy(edges_hbm.at[0, pl.ds(wid * CPW, GS)], src_v, ssem1)
    dc0 = pltpu.async_copy(edges_hbm.at[1, pl.ds(wid * CPW, GS)], dst_v, gsem1)
    sc0.wait()
    dc0.wait()
    pltpu.async_copy(h_hbm.at[src_v.at[0]], rows0_v, gsem0)
    zc.wait()
    plsc.subcore_barrier()

    for g in range(NGROUPS):
        if g > 0:
            pltpu.sync_copy(edges_hbm.at[0, pl.ds(wid * CPW + g * GS, GS)],
                            src_v)
            pltpu.sync_copy(edges_hbm.at[1, pl.ds(wid * CPW + g * GS, GS)],
                            dst_v)
            pltpu.async_copy(h_hbm.at[src_v.at[0]], rows0_v, gsem0)

        @pl.loop(0, GS, step=2)
        def _(j):
            for b in range(2):
                je = j + b
                pltpu.make_async_copy(h_hbm.at[src_v.at[je]], rows[b],
                                      gsems[b]).wait()
                nxt = je + 1

                @pl.when(nxt < GS)
                def _():
                    pltpu.async_copy(h_hbm.at[src_v.at[nxt]], rows[1 - b],
                                     gsems[1 - b])

                pltpu.sync_copy(rows[b], agg_sh.at[dst_v.at[je]], add=True)

    plsc.subcore_barrier()

    pltpu.sync_copy(agg_sh.at[pl.ds(s * RPS, RPS)],
                    out_hbm.at[c, pl.ds(s * RPS, RPS)])


@functools.cache
def _get_sc_agg():
    return pl.kernel(
        _sc_agg_body,
        out_type=jax.ShapeDtypeStruct((NC, NPAD, D), jnp.float32),
        mesh=plsc.VectorSubcoreMesh(core_axis_name="c", subcore_axis_name="s",
                                    num_cores=NC, num_subcores=NS),
        scratch_types=[
            pltpu.VMEM((GS, CHUNK), jnp.int32),
            pltpu.VMEM((GS, CHUNK), jnp.int32),
            pltpu.VMEM((CHUNK, D), jnp.float32),
            pltpu.VMEM((CHUNK, D), jnp.float32),
            pltpu.VMEM_SHARED((NPAD, D), jnp.float32),
            pltpu.SemaphoreType.DMA,
            pltpu.SemaphoreType.DMA,
            pltpu.SemaphoreType.DMA,
            pltpu.SemaphoreType.DMA,
        ],
    )


def _layer_body(h_ref, a_ref, w_ref, b_ref, o_ref):
    acc = h_ref[...] + a_ref[0] + a_ref[1]
    o_ref[...] = jnp.maximum(
        jnp.dot(acc, w_ref[...], preferred_element_type=jnp.float32)
        + b_ref[...], 0.0)


_tc_layer = pl.pallas_call(
    _layer_body,
    grid=(NB,),
    in_specs=[
        pl.BlockSpec((BLK, D), lambda i: (i, 0)),
        pl.BlockSpec((NC, BLK, D), lambda i: (0, i, 0)),
        pl.BlockSpec((D, D), lambda i: (0, 0)),
        pl.BlockSpec((1, D), lambda i: (0, 0)),
    ],
    out_specs=pl.BlockSpec((BLK, D), lambda i: (i, 0)),
    out_shape=jax.ShapeDtypeStruct((N, D), jnp.float32),
)


def _layer3_pool_body(h_ref, a_ref, w_ref, b_ref, batch_ref, wm_ref, bm_ref,
                      o_ref, acc_ref, cnt_ref):
    i = pl.program_id(0)

    @pl.when(i == 0)
    def _():
        acc_ref[...] = jnp.zeros_like(acc_ref)
        cnt_ref[...] = jnp.zeros_like(cnt_ref)

    acc = h_ref[...] + a_ref[0] + a_ref[1]
    h3 = jnp.maximum(
        jnp.dot(acc, w_ref[...], preferred_element_type=jnp.float32)
        + b_ref[...], 0.0)

    bvec = batch_ref[0, 0, :]
    seg = lax.broadcasted_iota(jnp.int32, (G, BLK), 0)
    onehot_t = (bvec[None, :] == seg).astype(jnp.float32)
    acc_ref[...] += jnp.dot(onehot_t, h3,
                            preferred_element_type=jnp.float32)
    cnt_ref[...] += jnp.sum(onehot_t, axis=1, keepdims=True)

    @pl.when(i == NB - 1)
    def _():
        pooled = acc_ref[...] / jnp.maximum(cnt_ref[...], 1.0)
        o_ref[...] = (jnp.dot(pooled, wm_ref[...],
                              preferred_element_type=jnp.float32)
                      + bm_ref[...])


_tc_layer3_pool = pl.pallas_call(
    _layer3_pool_body,
    grid=(NB,),
    in_specs=[
        pl.BlockSpec((BLK, D), lambda i: (i, 0)),
        pl.BlockSpec((NC, BLK, D), lambda i: (0, i, 0)),
        pl.BlockSpec((D, D), lambda i: (0, 0)),
        pl.BlockSpec((1, D), lambda i: (0, 0)),
        pl.BlockSpec((1, 1, BLK), lambda i: (i, 0, 0)),
        pl.BlockSpec((D, D), lambda i: (0, 0)),
        pl.BlockSpec((1, D), lambda i: (0, 0)),
    ],
    out_specs=pl.BlockSpec((G, D), lambda i: (0, 0)),
    out_shape=jax.ShapeDtypeStruct((G, D), jnp.float32),
    scratch_shapes=[
        pltpu.VMEM((G, D), jnp.float32),
        pltpu.VMEM((G, 1), jnp.float32),
    ],
)


def kernel(x, edge_index, batch, W1, b1, W2, b2, W3, b3, Wm, bm):
    edges3 = edge_index.reshape(2, ROWS2D, CHUNK)
    zeros = jnp.zeros((NPAD, D), jnp.float32)

    h = x
    for (W, b) in ((W1, b1), (W2, b2)):
        agg2 = _get_sc_agg()(h, edges3, zeros)
        h = _tc_layer(h, agg2, W, b.reshape(1, D))
    agg2 = _get_sc_agg()(h, edges3, zeros)
    return _tc_layer3_pool(h, agg2, W3, b3.reshape(1, D),
                           batch.reshape(NB, 1, BLK), Wm, bm.reshape(1, D))
